# baseline (speedup 1.0000x reference)
import jax
import jax.numpy as jnp
from jax.experimental import pallas as pl
from jax.experimental.pallas import tpu as pltpu
from jax.experimental.pallas import tpu_sc as plsc

_R = 8
_NBUF = 8


def _sc_gather(x2d, rows2d, n, d):
    mesh = plsc.ScalarSubcoreMesh(axis_name="core", num_cores=2)
    half = n // 2

    @jax.jit
    @pl.kernel(
        out_type=jax.ShapeDtypeStruct((n, d), x2d.dtype),
        mesh=mesh,
        scratch_types=[
            pltpu.SMEM((n,), jnp.int32),
            pltpu.SemaphoreType.DMA,
            pltpu.SemaphoreType.DMA,
        ],
    )
    def gather_kernel(x_hbm, i_hbm, o_hbm, idx_s, sem_i, sem_o):
        c = jax.lax.axis_index("core")
        pltpu.async_copy(i_hbm, idx_s, sem_i).wait()
        base = c * half

        @pl.loop(0, half)
        def _(k):
            i = base + k
            pltpu.async_copy(x_hbm.at[idx_s[i]], o_hbm.at[i], sem_o)

        @pl.loop(0, half)
        def _(k):
            pltpu.make_async_copy(x_hbm.at[0], o_hbm.at[0], sem_o).wait()

    return gather_kernel(x2d, rows2d)


def _tc_broadcast(compact, n, c, d):
    R, NBUF = _R, _NBUF
    nsteps = n // R

    def body(c_ref, o_hbm, scratch, sems):
        i = pl.program_id(0)
        slot = jax.lax.rem(i, NBUF)

        @pl.when(i >= NBUF)
        def _():
            pltpu.make_async_copy(
                scratch.at[slot],
                o_hbm.at[pl.ds((i - NBUF) * R, R)],
                sems.at[slot],
            ).wait()

        scratch[pl.ds(slot, 1)] = jnp.broadcast_to(
            c_ref[...][None, :, None, :], (1, R, c, d)
        )
        pltpu.make_async_copy(
            scratch.at[slot],
            o_hbm.at[pl.ds(i * R, R)],
            sems.at[slot],
        ).start()

        @pl.when(i == nsteps - 1)
        def _():
            for s in range(NBUF):
                j_s = i - jax.lax.rem(i - s, NBUF)
                pltpu.make_async_copy(
                    scratch.at[s],
                    o_hbm.at[pl.ds(j_s * R, R)],
                    sems.at[s],
                ).wait()

    out = pl.pallas_call(
        body,
        grid=(nsteps,),
        in_specs=[pl.BlockSpec((R, d), lambda i: (i, 0))],
        out_specs=pl.BlockSpec(memory_space=pl.ANY),
        out_shape=jax.ShapeDtypeStruct((n, c, d), compact.dtype),
        scratch_shapes=[
            pltpu.VMEM((NBUF, R, c, d), compact.dtype),
            pltpu.SemaphoreType.DMA((NBUF,)),
        ],
    )(compact)
    return out


def kernel(logits, indices):
    N, C, H, W = logits.shape
    D = H * W
    x2d = logits.reshape(N * C, D)
    idx = indices.astype(jnp.int32)
    rows2d = jnp.arange(N, dtype=jnp.int32) * C + idx

    compact = _sc_gather(x2d, rows2d, N, D)
    out = _tc_broadcast(compact, N, C, D)
    return out.reshape(N, C, H, W)
